# baseline (speedup 1.0000x reference)
import functools

import jax
import jax.numpy as jnp
from jax import lax
from jax.experimental import pallas as pl
from jax.experimental.pallas import tpu as pltpu
from jax.experimental.pallas import tpu_sc as plsc

_F32 = jnp.float32

_NC = 2
_NSUB = 16
_NW = _NC * _NSUB

_BE = 3200
_BN = 2000
_CH = 80


def _dot(a, b):
    return jnp.dot(a.astype(jnp.bfloat16), b.astype(jnp.bfloat16),
                   preferred_element_type=_F32)


def _msg_body(gsrc, gdst, es, ev0, ev1, ev2,
              w1ss, w1se, w1sd, w1svn, b1s, w1hs, w1he, w1hd, w1v, w1sv, b1sv,
              w2h, w2s, w2svn, b2s, w2v, w2sv, b2sv,
              w3h, w3s, w3svn, b3s, w3v, w3sv, b3sv,
              out_s, out_v):
    usrc = lax.bitcast_convert_type(gsrc[...], jnp.uint32)
    udst = lax.bitcast_convert_type(gdst[...], jnp.uint32)
    hi = jnp.uint32(0xFFFF0000)
    ssrc = lax.bitcast_convert_type(usrc & hi, _F32)
    sdst = lax.bitcast_convert_type(udst & hi, _F32)
    vsrc_all = lax.bitcast_convert_type(usrc << 16, _F32)
    vdst_all = lax.bitcast_convert_type(udst << 16, _F32)
    s_lin = _dot(ssrc, w1ss[...]) + _dot(es[...], w1se[...]) + \
        _dot(sdst, w1sd[...])
    vh = []
    for c, evc in enumerate((ev0, ev1, ev2)):
        vs = vsrc_all[:, 16 * c:16 * c + 16]
        vd = vdst_all[:, 16 * c:16 * c + 16]
        ev_outer = lax.dot_general(evc[...], w1he[...],
                                   (((0,), (0,)), ((), ())),
                                   preferred_element_type=_F32)
        vh.append(_dot(vs, w1hs[...]) + ev_outer + _dot(vd, w1hd[...]))
    vn = jnp.sqrt(jnp.clip(vh[0] * vh[0] + vh[1] * vh[1] + vh[2] * vh[2],
                           1e-8, None))
    s1 = s_lin + _dot(vn, w1svn[...]) + b1s[...]
    gate = jax.nn.sigmoid(_dot(jax.nn.sigmoid(s1), w1sv[...]) + b1sv[...])
    vo = [_dot(vh[c], w1v[...]) * gate for c in range(3)]
    s1 = jnp.maximum(s1, 0.0)
    vh = [_dot(vo[c], w2h[...]) for c in range(3)]
    vn = jnp.sqrt(jnp.clip(vh[0] * vh[0] + vh[1] * vh[1] + vh[2] * vh[2],
                           1e-8, None))
    s2 = _dot(s1, w2s[...]) + _dot(vn, w2svn[...]) + b2s[...]
    gate = jax.nn.sigmoid(_dot(jax.nn.sigmoid(s2), w2sv[...]) + b2sv[...])
    vo = [_dot(vh[c], w2v[...]) * gate for c in range(3)]
    s2 = jnp.maximum(s2, 0.0)
    vh = [_dot(vo[c], w3h[...]) for c in range(3)]
    vn = jnp.sqrt(jnp.clip(vh[0] * vh[0] + vh[1] * vh[1] + vh[2] * vh[2],
                           1e-8, None))
    s3 = _dot(s2, w3s[...]) + _dot(vn, w3svn[...]) + b3s[...]
    gate = jax.nn.sigmoid(_dot(s3, w3sv[...]) + b3sv[...])
    vo = [_dot(vh[c], w3v[...]) * gate for c in range(3)]
    out_s[...] = s3
    m = s3.shape[0]
    out_v[...] = jnp.concatenate(
        vo + [jnp.ones((m, 16), _F32), jnp.zeros((m, 64), _F32)], axis=1)


def _msg_weights(params):
    p1, p2, p3 = params['msg']
    ws1, wh1 = p1['Ws'], p1['Wh']
    w = [ws1[0:128], ws1[128:144], ws1[144:272], ws1[272:305],
         p1['bs'].reshape(1, -1), wh1[0:16], wh1[16:17], wh1[17:33],
         p1['Wv'], p1['Wsv'], p1['bsv'].reshape(1, -1)]
    for p in (p2, p3):
        w += [p['Wh'], p['Ws'][0:128], p['Ws'][128:144],
              p['bs'].reshape(1, -1), p['Wv'], p['Wsv'],
              p['bsv'].reshape(1, -1)]
    return [x if x.shape[0] == 1 else x.astype(jnp.bfloat16) for x in w]


def _run_msg(gsrc, gdst, es, evs, wlist):
    E = es.shape[0]
    grid = (E // _BE,)

    def _full(shape):
        return pl.BlockSpec(shape, lambda i: (0,) * len(shape))

    in_specs = [
        pl.BlockSpec((_BE, 128), lambda i: (i, 0)),
        pl.BlockSpec((_BE, 128), lambda i: (i, 0)),
        pl.BlockSpec((_BE, 16), lambda i: (i, 0)),
        pl.BlockSpec((1, _BE), lambda i: (0, i)),
        pl.BlockSpec((1, _BE), lambda i: (0, i)),
        pl.BlockSpec((1, _BE), lambda i: (0, i)),
    ] + [_full(x.shape) for x in wlist]
    out_specs = [pl.BlockSpec((_BE, 128), lambda i: (i, 0)),
                 pl.BlockSpec((_BE, 128), lambda i: (i, 0))]
    out_shape = [jax.ShapeDtypeStruct((E, 128), _F32),
                 jax.ShapeDtypeStruct((E, 128), _F32)]
    return pl.pallas_call(_msg_body, grid=grid, in_specs=in_specs,
                          out_specs=out_specs, out_shape=out_shape)(
                              gsrc, gdst, es, *evs, *wlist)


def _node_body(sums, sumvc, ns, nv, g0, t0, g1, t1,
               f1h, f1s, f1svn, f1bs, f1v, f1sv, f1bsv,
               f2h, f2s, f2svn, f2bs, f2v, f2sv, f2bsv,
               out_s, out_v):
    c = sumvc[0, :, 48:49] + sumvc[1, :, 48:49]
    inv = 1.0 / jnp.maximum(c, 1.0)
    s = ns[...] + (sums[0] + sums[1]) * inv
    v = nv[...] + (sumvc[0, :, 0:48] + sumvc[1, :, 0:48]) * inv
    mu = jnp.mean(s, axis=1, keepdims=True)
    var = jnp.mean((s - mu) * (s - mu), axis=1, keepdims=True)
    s = (s - mu) / jnp.sqrt(var + 1e-5) * g0[...] + t0[...]
    vnrm = jnp.sqrt(jnp.clip(
        jnp.sum(v * v, axis=1, keepdims=True) / 16.0, 1e-8, None))
    v = v / vnrm
    vc = [v[:, 16 * c:16 * (c + 1)] for c in range(3)]
    vh = [_dot(vc[c], f1h[...]) for c in range(3)]
    vn = jnp.sqrt(jnp.clip(vh[0] * vh[0] + vh[1] * vh[1] + vh[2] * vh[2],
                           1e-8, None))
    s1 = _dot(s, f1s[...]) + _dot(vn, f1svn[...]) + f1bs[...]
    gate = jax.nn.sigmoid(_dot(jax.nn.sigmoid(s1), f1sv[...]) + f1bsv[...])
    vo = [_dot(vh[c], f1v[...]) * gate for c in range(3)]
    s1 = jnp.maximum(s1, 0.0)
    vh = [_dot(vo[c], f2h[...]) for c in range(3)]
    vn = jnp.sqrt(jnp.clip(vh[0] * vh[0] + vh[1] * vh[1] + vh[2] * vh[2],
                           1e-8, None))
    s2 = _dot(s1, f2s[...]) + _dot(vn, f2svn[...]) + f2bs[...]
    gate = jax.nn.sigmoid(_dot(s2, f2sv[...]) + f2bsv[...])
    vo2 = [_dot(vh[c], f2v[...]) * gate for c in range(3)]
    so = s + s2
    vos = [vc[c] + vo2[c] for c in range(3)]
    mu = jnp.mean(so, axis=1, keepdims=True)
    var = jnp.mean((so - mu) * (so - mu), axis=1, keepdims=True)
    so = (so - mu) / jnp.sqrt(var + 1e-5) * g1[...] + t1[...]
    vsq = vos[0] * vos[0] + vos[1] * vos[1] + vos[2] * vos[2]
    vnrm = jnp.sqrt(jnp.clip(
        jnp.sum(vsq, axis=1, keepdims=True) / 16.0, 1e-8, None))
    out_s[...] = so
    out_v[...] = jnp.concatenate(vos, axis=1) / vnrm


def _node_weights(params):
    f1, f2 = params['ff']
    ln0, ln1 = params['ln']
    w = [ln0['gamma'].reshape(1, -1), ln0['beta'].reshape(1, -1),
         ln1['gamma'].reshape(1, -1), ln1['beta'].reshape(1, -1)]
    for p, si in ((f1, 128), (f2, 512)):
        w += [p['Wh'], p['Ws'][0:si], p['Ws'][si:],
              p['bs'].reshape(1, -1), p['Wv'], p['Wsv'],
              p['bsv'].reshape(1, -1)]
    return w


def _run_node(sums, sumvc, ns, nv, wlist):
    N = ns.shape[0]
    grid = (N // _BN,)

    def _full(shape):
        return pl.BlockSpec(shape, lambda i: (0,) * len(shape))

    in_specs = [
        pl.BlockSpec((2, _BN, 128), lambda i: (0, i, 0)),
        pl.BlockSpec((2, _BN, 128), lambda i: (0, i, 0)),
        pl.BlockSpec((_BN, 128), lambda i: (i, 0)),
        pl.BlockSpec((_BN, 48), lambda i: (i, 0)),
    ] + [_full(x.shape) for x in wlist]
    out_specs = [pl.BlockSpec((_BN, 128), lambda i: (i, 0)),
                 pl.BlockSpec((_BN, 48), lambda i: (i, 0))]
    out_shape = [jax.ShapeDtypeStruct((N, 128), _F32),
                 jax.ShapeDtypeStruct((N, 48), _F32)]
    return pl.pallas_call(_node_body, grid=grid, in_specs=in_specs,
                          out_specs=out_specs, out_shape=out_shape)(
                              sums, sumvc, ns, nv, *wlist)


def _sc_gather(packed, src, dst):
    E = src.shape[0]
    per_w = E // _NW
    n_chunks = per_w // _CH
    D = packed.shape[1]
    mesh = plsc.VectorSubcoreMesh(core_axis_name="c", subcore_axis_name="s")

    @functools.partial(
        pl.kernel, mesh=mesh,
        out_type=(jax.ShapeDtypeStruct((E, D), _F32),
                  jax.ShapeDtypeStruct((E, D), _F32)),
        scratch_types=[pltpu.VMEM((_CH,), jnp.int32),
                       pltpu.VMEM((_CH,), jnp.int32),
                       pltpu.VMEM((_CH, D), _F32),
                       pltpu.VMEM((_CH, D), _F32),
                       pltpu.SemaphoreType.DMA,
                       pltpu.SemaphoreType.DMA])
    def k(tab, src_h, dst_h, out_s, out_d, idx_s, idx_d, rows_s,
          rows_d, sem1, sem2):
        wid = lax.axis_index("s") * _NC + lax.axis_index("c")
        base = wid * per_w

        def body(i, carry):
            off = base + i * _CH
            pltpu.sync_copy(src_h.at[pl.ds(off, _CH)], idx_s)
            pltpu.sync_copy(dst_h.at[pl.ds(off, _CH)], idx_d)
            a = pltpu.async_copy(tab.at[idx_s], rows_s, sem1)
            b = pltpu.async_copy(tab.at[idx_d], rows_d, sem2)
            a.wait()
            b.wait()
            pltpu.sync_copy(rows_s, out_s.at[pl.ds(off, _CH)])
            pltpu.sync_copy(rows_d, out_d.at[pl.ds(off, _CH)])
            return carry

        lax.fori_loop(0, n_chunks, body, 0)

    return k(packed, src, dst)


def _sc_scatter(msg, dst, zeros):
    E = dst.shape[0]
    N = zeros.shape[0]
    per_w = E // _NW
    n_chunks = per_w // _CH
    mesh = plsc.VectorSubcoreMesh(core_axis_name="c", subcore_axis_name="s")

    @functools.partial(
        pl.kernel, mesh=mesh,
        out_type=jax.ShapeDtypeStruct((2, N, 128), _F32),
        scratch_types=[pltpu.VMEM((_CH,), jnp.int32),
                       pltpu.VMEM((_CH, 128), _F32),
                       pltpu.VMEM_SHARED((N, 128), _F32)])
    def k(msg_h, dst_h, z_h, out, idx, buf, acc):
        cid = lax.axis_index("c")
        sid = lax.axis_index("s")
        wid = sid * _NC + cid
        base = wid * per_w

        @pl.when(sid == 0)
        def _():
            pltpu.sync_copy(z_h, acc)

        plsc.subcore_barrier()

        def body(i, carry):
            off = base + i * _CH
            pltpu.sync_copy(dst_h.at[pl.ds(off, _CH)], idx)
            pltpu.sync_copy(msg_h.at[pl.ds(off, _CH)], buf)
            pltpu.sync_copy(buf, acc.at[idx], add=True)
            return carry

        lax.fori_loop(0, n_chunks, body, 0)
        plsc.subcore_barrier()

        @pl.when(sid == 0)
        def _():
            pltpu.sync_copy(acc, out.at[cid])

    return k(msg, dst, zeros)


def kernel(node_s, node_v, edge_s, edge_v, params, edge_index):
    N = node_s.shape[0]
    E = edge_s.shape[0]
    src = edge_index[0]
    dst = edge_index[1]
    nvt = node_v.swapaxes(1, 2).reshape(N, 48)
    vrow = jnp.concatenate([nvt, jnp.zeros((N, 80), _F32)], axis=1)
    sbits = lax.bitcast_convert_type(
        node_s.astype(jnp.bfloat16), jnp.uint16).astype(jnp.uint32)
    vbits = lax.bitcast_convert_type(
        vrow.astype(jnp.bfloat16), jnp.uint16).astype(jnp.uint32)
    packed = lax.bitcast_convert_type((sbits << 16) | vbits, _F32)
    evs = [edge_v[:, 0, c].reshape(1, E) for c in range(3)]

    gsrc, gdst = _sc_gather(packed, src, dst)
    msg_s, msg_vc = _run_msg(gsrc, gdst, edge_s, evs, _msg_weights(params))
    zeros = jnp.zeros((N, 128), _F32)
    sums = _sc_scatter(msg_s, dst, zeros)
    sumvc = _sc_scatter(msg_vc, dst, zeros)
    s_out, v48 = _run_node(sums, sumvc, node_s, nvt, _node_weights(params))
    v_out = v48.reshape(N, 3, 16).swapaxes(1, 2)
    return s_out, v_out


# speedup vs baseline: 1.4822x; 1.4822x over previous
import functools

import jax
import jax.numpy as jnp
from jax import lax
from jax.experimental import pallas as pl
from jax.experimental.pallas import tpu as pltpu
from jax.experimental.pallas import tpu_sc as plsc

_F32 = jnp.float32

_NC = 2
_NSUB = 16
_NW = _NC * _NSUB

_BE = 3200
_BN = 2000
_CH = 80


def _dot(a, b):
    return jnp.dot(a, b, preferred_element_type=_F32)


def _msg_body(gsrc, gdst, es, ev0, ev1, ev2,
              w1ss, w1se, w1sd, w1svn, b1s, w1hs, w1he, w1hd, w1v, w1sv, b1sv,
              w2h, w2s, w2svn, b2s, w2v, w2sv, b2sv,
              w3h, w3s, w3svn, b3s, w3v, w3sv, b3sv,
              out_s, out_v):
    usrc = lax.bitcast_convert_type(gsrc[...], jnp.uint32)
    udst = lax.bitcast_convert_type(gdst[...], jnp.uint32)
    hi = jnp.uint32(0xFFFF0000)
    ssrc = lax.bitcast_convert_type(usrc & hi, _F32)
    sdst = lax.bitcast_convert_type(udst & hi, _F32)
    vsrc_all = lax.bitcast_convert_type(usrc << 16, _F32)
    vdst_all = lax.bitcast_convert_type(udst << 16, _F32)
    s_lin = _dot(ssrc, w1ss[...]) + _dot(es[...], w1se[...]) + \
        _dot(sdst, w1sd[...])
    vh = []
    for c, evc in enumerate((ev0, ev1, ev2)):
        vs = vsrc_all[:, 16 * c:16 * c + 16]
        vd = vdst_all[:, 16 * c:16 * c + 16]
        ev_outer = lax.dot_general(evc[...], w1he[...],
                                   (((0,), (0,)), ((), ())),
                                   preferred_element_type=_F32)
        vh.append(_dot(vs, w1hs[...]) + ev_outer + _dot(vd, w1hd[...]))
    vn = jnp.sqrt(jnp.clip(vh[0] * vh[0] + vh[1] * vh[1] + vh[2] * vh[2],
                           1e-8, None))
    s1 = s_lin + _dot(vn, w1svn[...]) + b1s[...]
    gate = jax.nn.sigmoid(_dot(jax.nn.sigmoid(s1), w1sv[...]) + b1sv[...])
    vo = [_dot(vh[c], w1v[...]) * gate for c in range(3)]
    s1 = jnp.maximum(s1, 0.0)
    vh = [_dot(vo[c], w2h[...]) for c in range(3)]
    vn = jnp.sqrt(jnp.clip(vh[0] * vh[0] + vh[1] * vh[1] + vh[2] * vh[2],
                           1e-8, None))
    s2 = _dot(s1, w2s[...]) + _dot(vn, w2svn[...]) + b2s[...]
    gate = jax.nn.sigmoid(_dot(jax.nn.sigmoid(s2), w2sv[...]) + b2sv[...])
    vo = [_dot(vh[c], w2v[...]) * gate for c in range(3)]
    s2 = jnp.maximum(s2, 0.0)
    vh = [_dot(vo[c], w3h[...]) for c in range(3)]
    vn = jnp.sqrt(jnp.clip(vh[0] * vh[0] + vh[1] * vh[1] + vh[2] * vh[2],
                           1e-8, None))
    s3 = _dot(s2, w3s[...]) + _dot(vn, w3svn[...]) + b3s[...]
    gate = jax.nn.sigmoid(_dot(s3, w3sv[...]) + b3sv[...])
    vo = [_dot(vh[c], w3v[...]) * gate for c in range(3)]
    out_s[...] = s3
    m = s3.shape[0]
    out_v[...] = jnp.concatenate(
        vo + [jnp.ones((m, 16), _F32), jnp.zeros((m, 64), _F32)], axis=1)


def _msg_weights(params):
    p1, p2, p3 = params['msg']
    ws1, wh1 = p1['Ws'], p1['Wh']
    w = [ws1[0:128], ws1[128:144], ws1[144:272], ws1[272:305],
         p1['bs'].reshape(1, -1), wh1[0:16], wh1[16:17], wh1[17:33],
         p1['Wv'], p1['Wsv'], p1['bsv'].reshape(1, -1)]
    for p in (p2, p3):
        w += [p['Wh'], p['Ws'][0:128], p['Ws'][128:144],
              p['bs'].reshape(1, -1), p['Wv'], p['Wsv'],
              p['bsv'].reshape(1, -1)]
    return w


def _run_msg(gsrc, gdst, es, evs, wlist):
    E = es.shape[0]
    grid = (E // _BE,)

    def _full(shape):
        return pl.BlockSpec(shape, lambda i: (0,) * len(shape))

    in_specs = [
        pl.BlockSpec((_BE, 128), lambda i: (i, 0)),
        pl.BlockSpec((_BE, 128), lambda i: (i, 0)),
        pl.BlockSpec((_BE, 16), lambda i: (i, 0)),
        pl.BlockSpec((1, _BE), lambda i: (0, i)),
        pl.BlockSpec((1, _BE), lambda i: (0, i)),
        pl.BlockSpec((1, _BE), lambda i: (0, i)),
    ] + [_full(x.shape) for x in wlist]
    out_specs = [pl.BlockSpec((_BE, 128), lambda i: (i, 0)),
                 pl.BlockSpec((_BE, 128), lambda i: (i, 0))]
    out_shape = [jax.ShapeDtypeStruct((E, 128), _F32),
                 jax.ShapeDtypeStruct((E, 128), _F32)]
    return pl.pallas_call(_msg_body, grid=grid, in_specs=in_specs,
                          out_specs=out_specs, out_shape=out_shape)(
                              gsrc, gdst, es, *evs, *wlist)


def _node_math(tot_s, tot_vc, ns, nv, wrefs, out_s, out_v):
    (g0, t0, g1, t1,
     f1h, f1s, f1svn, f1bs, f1v, f1sv, f1bsv,
     f2h, f2s, f2svn, f2bs, f2v, f2sv, f2bsv) = wrefs
    c = tot_vc[:, 48:49]
    inv = 1.0 / jnp.maximum(c, 1.0)
    s = ns[...] + tot_s * inv
    v = nv[...] + tot_vc[:, 0:48] * inv
    mu = jnp.mean(s, axis=1, keepdims=True)
    var = jnp.mean((s - mu) * (s - mu), axis=1, keepdims=True)
    s = (s - mu) / jnp.sqrt(var + 1e-5) * g0[...] + t0[...]
    vnrm = jnp.sqrt(jnp.clip(
        jnp.sum(v * v, axis=1, keepdims=True) / 16.0, 1e-8, None))
    v = v / vnrm
    vc = [v[:, 16 * c:16 * (c + 1)] for c in range(3)]
    vh = [_dot(vc[c], f1h[...]) for c in range(3)]
    vn = jnp.sqrt(jnp.clip(vh[0] * vh[0] + vh[1] * vh[1] + vh[2] * vh[2],
                           1e-8, None))
    s1 = _dot(s, f1s[...]) + _dot(vn, f1svn[...]) + f1bs[...]
    gate = jax.nn.sigmoid(_dot(jax.nn.sigmoid(s1), f1sv[...]) + f1bsv[...])
    vo = [_dot(vh[c], f1v[...]) * gate for c in range(3)]
    s1 = jnp.maximum(s1, 0.0)
    vh = [_dot(vo[c], f2h[...]) for c in range(3)]
    vn = jnp.sqrt(jnp.clip(vh[0] * vh[0] + vh[1] * vh[1] + vh[2] * vh[2],
                           1e-8, None))
    s2 = _dot(s1, f2s[...]) + _dot(vn, f2svn[...]) + f2bs[...]
    gate = jax.nn.sigmoid(_dot(s2, f2sv[...]) + f2bsv[...])
    vo2 = [_dot(vh[c], f2v[...]) * gate for c in range(3)]
    so = s + s2
    vos = [vc[c] + vo2[c] for c in range(3)]
    mu = jnp.mean(so, axis=1, keepdims=True)
    var = jnp.mean((so - mu) * (so - mu), axis=1, keepdims=True)
    so = (so - mu) / jnp.sqrt(var + 1e-5) * g1[...] + t1[...]
    vsq = vos[0] * vos[0] + vos[1] * vos[1] + vos[2] * vos[2]
    vnrm = jnp.sqrt(jnp.clip(
        jnp.sum(vsq, axis=1, keepdims=True) / 16.0, 1e-8, None))
    out_s[...] = so
    out_v[...] = jnp.concatenate(vos, axis=1) / vnrm


def _node_weights(params):
    f1, f2 = params['ff']
    ln0, ln1 = params['ln']
    w = [ln0['gamma'].reshape(1, -1), ln0['beta'].reshape(1, -1),
         ln1['gamma'].reshape(1, -1), ln1['beta'].reshape(1, -1)]
    for p, si in ((f1, 128), (f2, 512)):
        w += [p['Wh'], p['Ws'][0:si], p['Ws'][si:],
              p['bs'].reshape(1, -1), p['Wv'], p['Wsv'],
              p['bsv'].reshape(1, -1)]
    return w


def _run_node(sums_list, sumvc_list, ns, nv, wlist):
    N = ns.shape[0]
    K = len(sums_list)
    grid = (N // _BN,)

    def _full(shape):
        return pl.BlockSpec(shape, lambda i: (0,) * len(shape))

    def body(*refs):
        sums = refs[0:K]
        sumvc = refs[K:2 * K]
        ns_r, nv_r = refs[2 * K], refs[2 * K + 1]
        wrefs = refs[2 * K + 2:-2]
        out_s, out_v = refs[-2], refs[-1]
        tot_s = sums[0][...]
        for r in sums[1:]:
            tot_s = tot_s + r[...]
        tot_vc = sumvc[0][...]
        for r in sumvc[1:]:
            tot_vc = tot_vc + r[...]
        _node_math(tot_s, tot_vc, ns_r, nv_r, wrefs, out_s, out_v)

    in_specs = (
        [pl.BlockSpec((_BN, 128), lambda i: (i, 0))] * (2 * K)
        + [pl.BlockSpec((_BN, 128), lambda i: (i, 0)),
           pl.BlockSpec((_BN, 48), lambda i: (i, 0))]
        + [_full(x.shape) for x in wlist])
    out_specs = [pl.BlockSpec((_BN, 128), lambda i: (i, 0)),
                 pl.BlockSpec((_BN, 48), lambda i: (i, 0))]
    out_shape = [jax.ShapeDtypeStruct((N, 128), _F32),
                 jax.ShapeDtypeStruct((N, 48), _F32)]
    return pl.pallas_call(body, grid=grid, in_specs=in_specs,
                          out_specs=out_specs, out_shape=out_shape)(
                              *sums_list, *sumvc_list, ns, nv, *wlist)


def _sc_gather(packed, src, dst):
    E = src.shape[0]
    per_w = E // _NW
    n_chunks = per_w // _CH
    D = packed.shape[1]
    mesh = plsc.VectorSubcoreMesh(core_axis_name="c", subcore_axis_name="s")

    @functools.partial(
        pl.kernel, mesh=mesh,
        out_type=(jax.ShapeDtypeStruct((E, D), _F32),
                  jax.ShapeDtypeStruct((E, D), _F32)),
        scratch_types=[pltpu.VMEM((_CH,), jnp.int32),
                       pltpu.VMEM((_CH,), jnp.int32),
                       pltpu.VMEM((_CH, D), _F32),
                       pltpu.VMEM((_CH, D), _F32),
                       pltpu.SemaphoreType.DMA,
                       pltpu.SemaphoreType.DMA])
    def k(tab, src_h, dst_h, out_s, out_d, idx_s, idx_d, rows_s,
          rows_d, sem1, sem2):
        wid = lax.axis_index("s") * _NC + lax.axis_index("c")
        base = wid * per_w

        def body(i, carry):
            off = base + i * _CH
            pltpu.sync_copy(src_h.at[pl.ds(off, _CH)], idx_s)
            pltpu.sync_copy(dst_h.at[pl.ds(off, _CH)], idx_d)
            a = pltpu.async_copy(tab.at[idx_s], rows_s, sem1)
            b = pltpu.async_copy(tab.at[idx_d], rows_d, sem2)
            a.wait()
            b.wait()
            pltpu.sync_copy(rows_s, out_s.at[pl.ds(off, _CH)])
            pltpu.sync_copy(rows_d, out_d.at[pl.ds(off, _CH)])
            return carry

        lax.fori_loop(0, n_chunks, body, 0)

    return k(packed, src, dst)


def _sc_scatter_pair(msg_a, msg_b, dst, zeros):
    E = dst.shape[0]
    N = zeros.shape[0]
    per_t = E // _NSUB
    n_chunks = per_t // _CH
    mesh = plsc.VectorSubcoreMesh(core_axis_name="c", subcore_axis_name="s")

    @functools.partial(
        pl.kernel, mesh=mesh,
        out_type=(jax.ShapeDtypeStruct((N, 128), _F32),
                  jax.ShapeDtypeStruct((N, 128), _F32)),
        scratch_types=[pltpu.VMEM((_CH,), jnp.int32),
                       pltpu.VMEM((_CH, 128), _F32),
                       pltpu.VMEM_SHARED((N, 128), _F32)])
    def k(ma_h, mb_h, dst_h, z_h, out_a, out_b, idx, buf, acc):
        cid = lax.axis_index("c")
        sid = lax.axis_index("s")
        base = sid * per_t

        @pl.when(sid == 0)
        def _():
            pltpu.sync_copy(z_h, acc)

        plsc.subcore_barrier()

        def body(i, carry):
            off = base + i * _CH
            pltpu.sync_copy(dst_h.at[pl.ds(off, _CH)], idx)

            @pl.when(cid == 0)
            def _():
                pltpu.sync_copy(ma_h.at[pl.ds(off, _CH)], buf)

            @pl.when(cid == 1)
            def _():
                pltpu.sync_copy(mb_h.at[pl.ds(off, _CH)], buf)

            pltpu.sync_copy(buf, acc.at[idx], add=True)
            return carry

        lax.fori_loop(0, n_chunks, body, 0)
        plsc.subcore_barrier()

        @pl.when((sid == 0) & (cid == 0))
        def _():
            pltpu.sync_copy(acc, out_a)

        @pl.when((sid == 0) & (cid == 1))
        def _():
            pltpu.sync_copy(acc, out_b)

    return k(msg_a, msg_b, dst, zeros)


def kernel(node_s, node_v, edge_s, edge_v, params, edge_index):
    N = node_s.shape[0]
    E = edge_s.shape[0]
    src = edge_index[0]
    dst = edge_index[1]
    nvt = node_v.swapaxes(1, 2).reshape(N, 48)
    vrow = jnp.concatenate([nvt, jnp.zeros((N, 80), _F32)], axis=1)
    sbits = lax.bitcast_convert_type(
        node_s.astype(jnp.bfloat16), jnp.uint16).astype(jnp.uint32)
    vbits = lax.bitcast_convert_type(
        vrow.astype(jnp.bfloat16), jnp.uint16).astype(jnp.uint32)
    packed = lax.bitcast_convert_type((sbits << 16) | vbits, _F32)
    evs = [edge_v[:, 0, c].reshape(1, E) for c in range(3)]

    K = 5
    Ec = E // K
    mw = _msg_weights(params)
    zeros = jnp.zeros((N, 128), _F32)
    sums_list, sumvc_list = [], []
    for k in range(K):
        lo, hi = k * Ec, (k + 1) * Ec
        gsrc, gdst = _sc_gather(packed, src[lo:hi], dst[lo:hi])
        msg_s, msg_vc = _run_msg(gsrc, gdst, edge_s[lo:hi],
                                 [e[:, lo:hi] for e in evs], mw)
        sk, vk = _sc_scatter_pair(msg_s, msg_vc, dst[lo:hi], zeros)
        sums_list.append(sk)
        sumvc_list.append(vk)
    s_out, v48 = _run_node(sums_list, sumvc_list, node_s, nvt,
                           _node_weights(params))
    v_out = v48.reshape(N, 3, 16).swapaxes(1, 2)
    return s_out, v_out


# speedup vs baseline: 1.5293x; 1.0318x over previous
import functools

import jax
import jax.numpy as jnp
from jax import lax
from jax.experimental import pallas as pl
from jax.experimental.pallas import tpu as pltpu
from jax.experimental.pallas import tpu_sc as plsc

_F32 = jnp.float32

_NC = 2
_NSUB = 16
_NW = _NC * _NSUB

_BE = 3200
_BN = 2000
_CH = 80


def _dot(a, b):
    return jnp.dot(a, b, preferred_element_type=_F32)


def _msg_body(gsrc, gdst, es, ev0, ev1, ev2,
              w1ss, w1se, w1sd, w1svn, b1s, w1hs, w1he, w1hd, w1v, w1sv, b1sv,
              w2h, w2s, w2svn, b2s, w2v, w2sv, b2sv,
              w3h, w3s, w3svn, b3s, w3v, w3sv, b3sv,
              out_s, out_v):
    usrc = lax.bitcast_convert_type(gsrc[...], jnp.uint32)
    udst = lax.bitcast_convert_type(gdst[...], jnp.uint32)
    hi = jnp.uint32(0xFFFF0000)
    ssrc = lax.bitcast_convert_type(usrc & hi, _F32)
    sdst = lax.bitcast_convert_type(udst & hi, _F32)
    vsrc_all = lax.bitcast_convert_type(usrc << 16, _F32)
    vdst_all = lax.bitcast_convert_type(udst << 16, _F32)
    s_lin = _dot(ssrc, w1ss[...]) + _dot(es[...], w1se[...]) + \
        _dot(sdst, w1sd[...])
    vh = []
    for c, evc in enumerate((ev0, ev1, ev2)):
        vs = vsrc_all[:, 16 * c:16 * c + 16]
        vd = vdst_all[:, 16 * c:16 * c + 16]
        ev_outer = lax.dot_general(evc[...], w1he[...],
                                   (((0,), (0,)), ((), ())),
                                   preferred_element_type=_F32)
        vh.append(_dot(vs, w1hs[...]) + ev_outer + _dot(vd, w1hd[...]))
    vn = jnp.sqrt(jnp.clip(vh[0] * vh[0] + vh[1] * vh[1] + vh[2] * vh[2],
                           1e-8, None))
    s1 = s_lin + _dot(vn, w1svn[...]) + b1s[...]
    gate = jax.nn.sigmoid(_dot(jax.nn.sigmoid(s1), w1sv[...]) + b1sv[...])
    vo = [_dot(vh[c], w1v[...]) * gate for c in range(3)]
    s1 = jnp.maximum(s1, 0.0)
    vh = [_dot(vo[c], w2h[...]) for c in range(3)]
    vn = jnp.sqrt(jnp.clip(vh[0] * vh[0] + vh[1] * vh[1] + vh[2] * vh[2],
                           1e-8, None))
    s2 = _dot(s1, w2s[...]) + _dot(vn, w2svn[...]) + b2s[...]
    gate = jax.nn.sigmoid(_dot(jax.nn.sigmoid(s2), w2sv[...]) + b2sv[...])
    vo = [_dot(vh[c], w2v[...]) * gate for c in range(3)]
    s2 = jnp.maximum(s2, 0.0)
    vh = [_dot(vo[c], w3h[...]) for c in range(3)]
    vn = jnp.sqrt(jnp.clip(vh[0] * vh[0] + vh[1] * vh[1] + vh[2] * vh[2],
                           1e-8, None))
    s3 = _dot(s2, w3s[...]) + _dot(vn, w3svn[...]) + b3s[...]
    gate = jax.nn.sigmoid(_dot(s3, w3sv[...]) + b3sv[...])
    vo = [_dot(vh[c], w3v[...]) * gate for c in range(3)]
    out_s[...] = s3
    m = s3.shape[0]
    out_v[...] = jnp.concatenate(
        vo + [jnp.ones((m, 16), _F32), jnp.zeros((m, 64), _F32)], axis=1)


def _msg_weights(params):
    p1, p2, p3 = params['msg']
    ws1, wh1 = p1['Ws'], p1['Wh']
    w = [ws1[0:128], ws1[128:144], ws1[144:272], ws1[272:305],
         p1['bs'].reshape(1, -1), wh1[0:16], wh1[16:17], wh1[17:33],
         p1['Wv'], p1['Wsv'], p1['bsv'].reshape(1, -1)]
    for p in (p2, p3):
        w += [p['Wh'], p['Ws'][0:128], p['Ws'][128:144],
              p['bs'].reshape(1, -1), p['Wv'], p['Wsv'],
              p['bsv'].reshape(1, -1)]
    return w


def _run_msg(gsrc, gdst, es, evs, wlist):
    E = es.shape[0]
    grid = (E // _BE,)

    def _full(shape):
        return pl.BlockSpec(shape, lambda i: (0,) * len(shape))

    in_specs = [
        pl.BlockSpec((_BE, 128), lambda i: (i, 0)),
        pl.BlockSpec((_BE, 128), lambda i: (i, 0)),
        pl.BlockSpec((_BE, 16), lambda i: (i, 0)),
        pl.BlockSpec((1, _BE), lambda i: (0, i)),
        pl.BlockSpec((1, _BE), lambda i: (0, i)),
        pl.BlockSpec((1, _BE), lambda i: (0, i)),
    ] + [_full(x.shape) for x in wlist]
    out_specs = [pl.BlockSpec((_BE, 128), lambda i: (i, 0)),
                 pl.BlockSpec((_BE, 128), lambda i: (i, 0))]
    out_shape = [jax.ShapeDtypeStruct((E, 128), _F32),
                 jax.ShapeDtypeStruct((E, 128), _F32)]
    return pl.pallas_call(_msg_body, grid=grid, in_specs=in_specs,
                          out_specs=out_specs, out_shape=out_shape)(
                              gsrc, gdst, es, *evs, *wlist)


def _node_math(tot_s, tot_vc, ns, nv, wrefs, out_s, out_v):
    (g0, t0, g1, t1,
     f1h, f1s, f1svn, f1bs, f1v, f1sv, f1bsv,
     f2h, f2s, f2svn, f2bs, f2v, f2sv, f2bsv) = wrefs
    c = tot_vc[:, 48:49]
    inv = 1.0 / jnp.maximum(c, 1.0)
    s = ns[...] + tot_s * inv
    v = nv[...] + tot_vc[:, 0:48] * inv
    mu = jnp.mean(s, axis=1, keepdims=True)
    var = jnp.mean((s - mu) * (s - mu), axis=1, keepdims=True)
    s = (s - mu) / jnp.sqrt(var + 1e-5) * g0[...] + t0[...]
    vnrm = jnp.sqrt(jnp.clip(
        jnp.sum(v * v, axis=1, keepdims=True) / 16.0, 1e-8, None))
    v = v / vnrm
    vc = [v[:, 16 * c:16 * (c + 1)] for c in range(3)]
    vh = [_dot(vc[c], f1h[...]) for c in range(3)]
    vn = jnp.sqrt(jnp.clip(vh[0] * vh[0] + vh[1] * vh[1] + vh[2] * vh[2],
                           1e-8, None))
    s1 = _dot(s, f1s[...]) + _dot(vn, f1svn[...]) + f1bs[...]
    gate = jax.nn.sigmoid(_dot(jax.nn.sigmoid(s1), f1sv[...]) + f1bsv[...])
    vo = [_dot(vh[c], f1v[...]) * gate for c in range(3)]
    s1 = jnp.maximum(s1, 0.0)
    vh = [_dot(vo[c], f2h[...]) for c in range(3)]
    vn = jnp.sqrt(jnp.clip(vh[0] * vh[0] + vh[1] * vh[1] + vh[2] * vh[2],
                           1e-8, None))
    s2 = _dot(s1, f2s[...]) + _dot(vn, f2svn[...]) + f2bs[...]
    gate = jax.nn.sigmoid(_dot(s2, f2sv[...]) + f2bsv[...])
    vo2 = [_dot(vh[c], f2v[...]) * gate for c in range(3)]
    so = s + s2
    vos = [vc[c] + vo2[c] for c in range(3)]
    mu = jnp.mean(so, axis=1, keepdims=True)
    var = jnp.mean((so - mu) * (so - mu), axis=1, keepdims=True)
    so = (so - mu) / jnp.sqrt(var + 1e-5) * g1[...] + t1[...]
    vsq = vos[0] * vos[0] + vos[1] * vos[1] + vos[2] * vos[2]
    vnrm = jnp.sqrt(jnp.clip(
        jnp.sum(vsq, axis=1, keepdims=True) / 16.0, 1e-8, None))
    out_s[...] = so
    out_v[...] = jnp.concatenate(vos, axis=1) / vnrm


def _node_weights(params):
    f1, f2 = params['ff']
    ln0, ln1 = params['ln']
    w = [ln0['gamma'].reshape(1, -1), ln0['beta'].reshape(1, -1),
         ln1['gamma'].reshape(1, -1), ln1['beta'].reshape(1, -1)]
    for p, si in ((f1, 128), (f2, 512)):
        w += [p['Wh'], p['Ws'][0:si], p['Ws'][si:],
              p['bs'].reshape(1, -1), p['Wv'], p['Wsv'],
              p['bsv'].reshape(1, -1)]
    return w


def _run_node(sums_list, sumvc_list, ns, nv, wlist):
    N = ns.shape[0]
    K = len(sums_list)
    grid = (N // _BN,)

    def _full(shape):
        return pl.BlockSpec(shape, lambda i: (0,) * len(shape))

    def body(*refs):
        sums = refs[0:K]
        sumvc = refs[K:2 * K]
        ns_r, nv_r = refs[2 * K], refs[2 * K + 1]
        wrefs = refs[2 * K + 2:-2]
        out_s, out_v = refs[-2], refs[-1]
        tot_s = sums[0][...]
        for r in sums[1:]:
            tot_s = tot_s + r[...]
        tot_vc = sumvc[0][...]
        for r in sumvc[1:]:
            tot_vc = tot_vc + r[...]
        _node_math(tot_s, tot_vc, ns_r, nv_r, wrefs, out_s, out_v)

    in_specs = (
        [pl.BlockSpec((_BN, 128), lambda i: (i, 0))] * (2 * K)
        + [pl.BlockSpec((_BN, 128), lambda i: (i, 0)),
           pl.BlockSpec((_BN, 48), lambda i: (i, 0))]
        + [_full(x.shape) for x in wlist])
    out_specs = [pl.BlockSpec((_BN, 128), lambda i: (i, 0)),
                 pl.BlockSpec((_BN, 48), lambda i: (i, 0))]
    out_shape = [jax.ShapeDtypeStruct((N, 128), _F32),
                 jax.ShapeDtypeStruct((N, 48), _F32)]
    return pl.pallas_call(body, grid=grid, in_specs=in_specs,
                          out_specs=out_specs, out_shape=out_shape)(
                              *sums_list, *sumvc_list, ns, nv, *wlist)


def _sc_gather(packed, src, dst):
    E = src.shape[0]
    per_w = E // _NW
    n_chunks = per_w // _CH
    D = packed.shape[1]
    mesh = plsc.VectorSubcoreMesh(core_axis_name="c", subcore_axis_name="s")

    @functools.partial(
        pl.kernel, mesh=mesh,
        out_type=(jax.ShapeDtypeStruct((E, D), _F32),
                  jax.ShapeDtypeStruct((E, D), _F32)),
        scratch_types=[pltpu.VMEM((_CH,), jnp.int32),
                       pltpu.VMEM((_CH,), jnp.int32),
                       pltpu.VMEM((_CH, D), _F32),
                       pltpu.VMEM((_CH, D), _F32),
                       pltpu.SemaphoreType.DMA,
                       pltpu.SemaphoreType.DMA])
    def k(tab, src_h, dst_h, out_s, out_d, idx_s, idx_d, rows_s,
          rows_d, sem1, sem2):
        wid = lax.axis_index("s") * _NC + lax.axis_index("c")
        base = wid * per_w

        def body(i, carry):
            off = base + i * _CH
            pltpu.sync_copy(src_h.at[pl.ds(off, _CH)], idx_s)
            pltpu.sync_copy(dst_h.at[pl.ds(off, _CH)], idx_d)
            a = pltpu.async_copy(tab.at[idx_s], rows_s, sem1)
            b = pltpu.async_copy(tab.at[idx_d], rows_d, sem2)
            a.wait()
            b.wait()
            pltpu.sync_copy(rows_s, out_s.at[pl.ds(off, _CH)])
            pltpu.sync_copy(rows_d, out_d.at[pl.ds(off, _CH)])
            return carry

        lax.fori_loop(0, n_chunks, body, 0)

    return k(packed, src, dst)


def _sc_scatter_pair(msg_a, msg_b, dst, zeros):
    E = dst.shape[0]
    N = zeros.shape[0]
    per_t = E // _NSUB
    n_chunks = per_t // _CH
    mesh = plsc.VectorSubcoreMesh(core_axis_name="c", subcore_axis_name="s")

    assert n_chunks % 2 == 0

    @functools.partial(
        pl.kernel, mesh=mesh,
        out_type=(jax.ShapeDtypeStruct((N, 128), _F32),
                  jax.ShapeDtypeStruct((N, 128), _F32)),
        scratch_types=[pltpu.VMEM((_CH,), jnp.int32),
                       pltpu.VMEM((_CH,), jnp.int32),
                       pltpu.VMEM((_CH, 128), _F32),
                       pltpu.VMEM((_CH, 128), _F32),
                       pltpu.SemaphoreType.DMA,
                       pltpu.SemaphoreType.DMA,
                       pltpu.SemaphoreType.DMA,
                       pltpu.SemaphoreType.DMA,
                       pltpu.VMEM_SHARED((N, 128), _F32)])
    def k(ma_h, mb_h, dst_h, z_h, out_a, out_b, idx0, idx1, buf0, buf1,
          si0, sb0, si1, sb1, acc):
        cid = lax.axis_index("c")
        sid = lax.axis_index("s")
        base = sid * per_t

        @pl.when(sid == 0)
        def _():
            pltpu.sync_copy(z_h, acc)

        plsc.subcore_barrier()

        def start(i, idx, buf, si, sb):
            off = base + i * _CH
            pltpu.async_copy(dst_h.at[pl.ds(off, _CH)], idx, si)

            @pl.when(cid == 0)
            def _():
                pltpu.async_copy(ma_h.at[pl.ds(off, _CH)], buf, sb)

            @pl.when(cid == 1)
            def _():
                pltpu.async_copy(mb_h.at[pl.ds(off, _CH)], buf, sb)

        def drain_and_scatter(idx, buf, si, sb):
            pltpu.make_async_copy(dst_h.at[pl.ds(base, _CH)], idx, si).wait()
            pltpu.make_async_copy(ma_h.at[pl.ds(base, _CH)], buf, sb).wait()
            pltpu.sync_copy(buf, acc.at[idx], add=True)

        start(0, idx0, buf0, si0, sb0)

        def body(j, carry):
            i = j * 2
            start(i + 1, idx1, buf1, si1, sb1)
            drain_and_scatter(idx0, buf0, si0, sb0)

            @pl.when(i + 2 < n_chunks)
            def _():
                start(i + 2, idx0, buf0, si0, sb0)

            drain_and_scatter(idx1, buf1, si1, sb1)
            return carry

        lax.fori_loop(0, n_chunks // 2, body, 0)
        plsc.subcore_barrier()

        @pl.when((sid == 0) & (cid == 0))
        def _():
            pltpu.sync_copy(acc, out_a)

        @pl.when((sid == 0) & (cid == 1))
        def _():
            pltpu.sync_copy(acc, out_b)

    return k(msg_a, msg_b, dst, zeros)


def kernel(node_s, node_v, edge_s, edge_v, params, edge_index):
    N = node_s.shape[0]
    E = edge_s.shape[0]
    src = edge_index[0]
    dst = edge_index[1]
    nvt = node_v.swapaxes(1, 2).reshape(N, 48)
    vrow = jnp.concatenate([nvt, jnp.zeros((N, 80), _F32)], axis=1)
    sbits = lax.bitcast_convert_type(
        node_s.astype(jnp.bfloat16), jnp.uint16).astype(jnp.uint32)
    vbits = lax.bitcast_convert_type(
        vrow.astype(jnp.bfloat16), jnp.uint16).astype(jnp.uint32)
    packed = lax.bitcast_convert_type((sbits << 16) | vbits, _F32)
    evs = [edge_v[:, 0, c].reshape(1, E) for c in range(3)]

    K = 5
    Ec = E // K
    mw = _msg_weights(params)
    zeros = jnp.zeros((N, 128), _F32)
    sums_list, sumvc_list = [], []
    for k in range(K):
        lo, hi = k * Ec, (k + 1) * Ec
        gsrc, gdst = _sc_gather(packed, src[lo:hi], dst[lo:hi])
        msg_s, msg_vc = _run_msg(gsrc, gdst, edge_s[lo:hi],
                                 [e[:, lo:hi] for e in evs], mw)
        sk, vk = _sc_scatter_pair(msg_s, msg_vc, dst[lo:hi], zeros)
        sums_list.append(sk)
        sumvc_list.append(vk)
    s_out, v48 = _run_node(sums_list, sumvc_list, node_s, nvt,
                           _node_weights(params))
    v_out = v48.reshape(N, 3, 16).swapaxes(1, 2)
    return s_out, v_out


# speedup vs baseline: 1.5661x; 1.0240x over previous
import functools

import jax
import jax.numpy as jnp
from jax import lax
from jax.experimental import pallas as pl
from jax.experimental.pallas import tpu as pltpu
from jax.experimental.pallas import tpu_sc as plsc

_F32 = jnp.float32

_NC = 2
_NSUB = 16
_NW = _NC * _NSUB

_BE = 3200
_BN = 2000
_CH = 80


def _dot(a, b):
    return jnp.dot(a, b, preferred_element_type=_F32)


def _msg_body(gsrc, gdst, es, ev0, ev1, ev2,
              w1ss, w1se, w1sd, w1svn, b1s, w1hs, w1he, w1hd, w1v, w1sv, b1sv,
              w2h, w2s, w2svn, b2s, w2v, w2sv, b2sv,
              w3h, w3s, w3svn, b3s, w3v, w3sv, b3sv,
              out_s, out_v):
    usrc = lax.bitcast_convert_type(gsrc[...], jnp.uint32)
    udst = lax.bitcast_convert_type(gdst[...], jnp.uint32)
    hi = jnp.uint32(0xFFFF0000)
    ssrc = lax.bitcast_convert_type(usrc & hi, _F32)
    sdst = lax.bitcast_convert_type(udst & hi, _F32)
    vsrc_all = lax.bitcast_convert_type(usrc << 16, _F32)
    vdst_all = lax.bitcast_convert_type(udst << 16, _F32)
    s_lin = _dot(ssrc, w1ss[...]) + _dot(es[...], w1se[...]) + \
        _dot(sdst, w1sd[...])
    vh = []
    for c, evc in enumerate((ev0, ev1, ev2)):
        vs = vsrc_all[:, 16 * c:16 * c + 16]
        vd = vdst_all[:, 16 * c:16 * c + 16]
        ev_outer = lax.dot_general(evc[...], w1he[...],
                                   (((0,), (0,)), ((), ())),
                                   preferred_element_type=_F32)
        vh.append(_dot(vs, w1hs[...]) + ev_outer + _dot(vd, w1hd[...]))
    vn = jnp.sqrt(jnp.clip(vh[0] * vh[0] + vh[1] * vh[1] + vh[2] * vh[2],
                           1e-8, None))
    s1 = s_lin + _dot(vn, w1svn[...]) + b1s[...]
    gate = jax.nn.sigmoid(_dot(jax.nn.sigmoid(s1), w1sv[...]) + b1sv[...])
    vo = [_dot(vh[c], w1v[...]) * gate for c in range(3)]
    s1 = jnp.maximum(s1, 0.0)
    vh = [_dot(vo[c], w2h[...]) for c in range(3)]
    vn = jnp.sqrt(jnp.clip(vh[0] * vh[0] + vh[1] * vh[1] + vh[2] * vh[2],
                           1e-8, None))
    s2 = _dot(s1, w2s[...]) + _dot(vn, w2svn[...]) + b2s[...]
    gate = jax.nn.sigmoid(_dot(jax.nn.sigmoid(s2), w2sv[...]) + b2sv[...])
    vo = [_dot(vh[c], w2v[...]) * gate for c in range(3)]
    s2 = jnp.maximum(s2, 0.0)
    vh = [_dot(vo[c], w3h[...]) for c in range(3)]
    vn = jnp.sqrt(jnp.clip(vh[0] * vh[0] + vh[1] * vh[1] + vh[2] * vh[2],
                           1e-8, None))
    s3 = _dot(s2, w3s[...]) + _dot(vn, w3svn[...]) + b3s[...]
    gate = jax.nn.sigmoid(_dot(s3, w3sv[...]) + b3sv[...])
    vo = [_dot(vh[c], w3v[...]) * gate for c in range(3)]
    out_s[...] = s3
    m = s3.shape[0]
    out_v[...] = jnp.concatenate(
        vo + [jnp.ones((m, 16), _F32), jnp.zeros((m, 64), _F32)], axis=1)


def _msg_weights(params):
    p1, p2, p3 = params['msg']
    ws1, wh1 = p1['Ws'], p1['Wh']
    w = [ws1[0:128], ws1[128:144], ws1[144:272], ws1[272:305],
         p1['bs'].reshape(1, -1), wh1[0:16], wh1[16:17], wh1[17:33],
         p1['Wv'], p1['Wsv'], p1['bsv'].reshape(1, -1)]
    for p in (p2, p3):
        w += [p['Wh'], p['Ws'][0:128], p['Ws'][128:144],
              p['bs'].reshape(1, -1), p['Wv'], p['Wsv'],
              p['bsv'].reshape(1, -1)]
    return w


def _run_msg(gsrc, gdst, es, evs, wlist):
    E = es.shape[0]
    grid = (E // _BE,)

    def _full(shape):
        return pl.BlockSpec(shape, lambda i: (0,) * len(shape))

    in_specs = [
        pl.BlockSpec((_BE, 128), lambda i: (i, 0)),
        pl.BlockSpec((_BE, 128), lambda i: (i, 0)),
        pl.BlockSpec((_BE, 16), lambda i: (i, 0)),
        pl.BlockSpec((1, _BE), lambda i: (0, i)),
        pl.BlockSpec((1, _BE), lambda i: (0, i)),
        pl.BlockSpec((1, _BE), lambda i: (0, i)),
    ] + [_full(x.shape) for x in wlist]
    out_specs = [pl.BlockSpec((_BE, 128), lambda i: (i, 0)),
                 pl.BlockSpec((_BE, 128), lambda i: (i, 0))]
    out_shape = [jax.ShapeDtypeStruct((E, 128), _F32),
                 jax.ShapeDtypeStruct((E, 128), _F32)]
    return pl.pallas_call(_msg_body, grid=grid, in_specs=in_specs,
                          out_specs=out_specs, out_shape=out_shape)(
                              gsrc, gdst, es, *evs, *wlist)


def _node_math(tot_s, tot_vc, ns, nv, wrefs, out_s, out_v):
    (g0, t0, g1, t1,
     f1h, f1s, f1svn, f1bs, f1v, f1sv, f1bsv,
     f2h, f2s, f2svn, f2bs, f2v, f2sv, f2bsv) = wrefs
    c = tot_vc[:, 48:49]
    inv = 1.0 / jnp.maximum(c, 1.0)
    s = ns[...] + tot_s * inv
    v = nv[...] + tot_vc[:, 0:48] * inv
    mu = jnp.mean(s, axis=1, keepdims=True)
    var = jnp.mean((s - mu) * (s - mu), axis=1, keepdims=True)
    s = (s - mu) / jnp.sqrt(var + 1e-5) * g0[...] + t0[...]
    vnrm = jnp.sqrt(jnp.clip(
        jnp.sum(v * v, axis=1, keepdims=True) / 16.0, 1e-8, None))
    v = v / vnrm
    vc = [v[:, 16 * c:16 * (c + 1)] for c in range(3)]
    vh = [_dot(vc[c], f1h[...]) for c in range(3)]
    vn = jnp.sqrt(jnp.clip(vh[0] * vh[0] + vh[1] * vh[1] + vh[2] * vh[2],
                           1e-8, None))
    s1 = _dot(s, f1s[...]) + _dot(vn, f1svn[...]) + f1bs[...]
    gate = jax.nn.sigmoid(_dot(jax.nn.sigmoid(s1), f1sv[...]) + f1bsv[...])
    vo = [_dot(vh[c], f1v[...]) * gate for c in range(3)]
    s1 = jnp.maximum(s1, 0.0)
    vh = [_dot(vo[c], f2h[...]) for c in range(3)]
    vn = jnp.sqrt(jnp.clip(vh[0] * vh[0] + vh[1] * vh[1] + vh[2] * vh[2],
                           1e-8, None))
    s2 = _dot(s1, f2s[...]) + _dot(vn, f2svn[...]) + f2bs[...]
    gate = jax.nn.sigmoid(_dot(s2, f2sv[...]) + f2bsv[...])
    vo2 = [_dot(vh[c], f2v[...]) * gate for c in range(3)]
    so = s + s2
    vos = [vc[c] + vo2[c] for c in range(3)]
    mu = jnp.mean(so, axis=1, keepdims=True)
    var = jnp.mean((so - mu) * (so - mu), axis=1, keepdims=True)
    so = (so - mu) / jnp.sqrt(var + 1e-5) * g1[...] + t1[...]
    vsq = vos[0] * vos[0] + vos[1] * vos[1] + vos[2] * vos[2]
    vnrm = jnp.sqrt(jnp.clip(
        jnp.sum(vsq, axis=1, keepdims=True) / 16.0, 1e-8, None))
    out_s[...] = so
    out_v[...] = jnp.concatenate(vos, axis=1) / vnrm


def _node_weights(params):
    f1, f2 = params['ff']
    ln0, ln1 = params['ln']
    w = [ln0['gamma'].reshape(1, -1), ln0['beta'].reshape(1, -1),
         ln1['gamma'].reshape(1, -1), ln1['beta'].reshape(1, -1)]
    for p, si in ((f1, 128), (f2, 512)):
        w += [p['Wh'], p['Ws'][0:si], p['Ws'][si:],
              p['bs'].reshape(1, -1), p['Wv'], p['Wsv'],
              p['bsv'].reshape(1, -1)]
    return w


def _run_node(sums_list, sumvc_list, ns, nv, wlist):
    N = ns.shape[0]
    K = len(sums_list)
    grid = (N // _BN,)

    def _full(shape):
        return pl.BlockSpec(shape, lambda i: (0,) * len(shape))

    def body(*refs):
        sums = refs[0:K]
        sumvc = refs[K:2 * K]
        ns_r, nv_r = refs[2 * K], refs[2 * K + 1]
        wrefs = refs[2 * K + 2:-2]
        out_s, out_v = refs[-2], refs[-1]
        tot_s = sums[0][...]
        for r in sums[1:]:
            tot_s = tot_s + r[...]
        tot_vc = sumvc[0][...]
        for r in sumvc[1:]:
            tot_vc = tot_vc + r[...]
        _node_math(tot_s, tot_vc, ns_r, nv_r, wrefs, out_s, out_v)

    in_specs = (
        [pl.BlockSpec((_BN, 128), lambda i: (i, 0))] * (2 * K)
        + [pl.BlockSpec((_BN, 128), lambda i: (i, 0)),
           pl.BlockSpec((_BN, 48), lambda i: (i, 0))]
        + [_full(x.shape) for x in wlist])
    out_specs = [pl.BlockSpec((_BN, 128), lambda i: (i, 0)),
                 pl.BlockSpec((_BN, 48), lambda i: (i, 0))]
    out_shape = [jax.ShapeDtypeStruct((N, 128), _F32),
                 jax.ShapeDtypeStruct((N, 48), _F32)]
    return pl.pallas_call(body, grid=grid, in_specs=in_specs,
                          out_specs=out_specs, out_shape=out_shape)(
                              *sums_list, *sumvc_list, ns, nv, *wlist)


def _sc_gather(packed, src, dst):
    E = src.shape[0]
    per_w = E // _NW
    n_chunks = per_w // _CH
    D = packed.shape[1]
    mesh = plsc.VectorSubcoreMesh(core_axis_name="c", subcore_axis_name="s")

    @functools.partial(
        pl.kernel, mesh=mesh,
        out_type=(jax.ShapeDtypeStruct((E, D), _F32),
                  jax.ShapeDtypeStruct((E, D), _F32)),
        scratch_types=[pltpu.VMEM((_CH,), jnp.int32),
                       pltpu.VMEM((_CH,), jnp.int32),
                       pltpu.VMEM((_CH, D), _F32),
                       pltpu.VMEM((_CH, D), _F32),
                       pltpu.SemaphoreType.DMA,
                       pltpu.SemaphoreType.DMA])
    def k(tab, src_h, dst_h, out_s, out_d, idx_s, idx_d, rows_s,
          rows_d, sem1, sem2):
        wid = lax.axis_index("s") * _NC + lax.axis_index("c")
        base = wid * per_w

        def body(i, carry):
            off = base + i * _CH
            pltpu.sync_copy(src_h.at[pl.ds(off, _CH)], idx_s)
            pltpu.sync_copy(dst_h.at[pl.ds(off, _CH)], idx_d)
            a = pltpu.async_copy(tab.at[idx_s], rows_s, sem1)
            b = pltpu.async_copy(tab.at[idx_d], rows_d, sem2)
            a.wait()
            b.wait()
            pltpu.sync_copy(rows_s, out_s.at[pl.ds(off, _CH)])
            pltpu.sync_copy(rows_d, out_d.at[pl.ds(off, _CH)])
            return carry

        lax.fori_loop(0, n_chunks, body, 0)

    return k(packed, src, dst)


def _sc_scatter_pair(msg_a, msg_b, dst, zeros):
    E = dst.shape[0]
    N = zeros.shape[0]
    per_t = E // _NSUB
    n_chunks = per_t // _CH
    mesh = plsc.VectorSubcoreMesh(core_axis_name="c", subcore_axis_name="s")

    assert n_chunks % 2 == 0

    @functools.partial(
        pl.kernel, mesh=mesh,
        out_type=(jax.ShapeDtypeStruct((N, 128), _F32),
                  jax.ShapeDtypeStruct((N, 128), _F32)),
        scratch_types=[pltpu.VMEM((_CH,), jnp.int32),
                       pltpu.VMEM((_CH,), jnp.int32),
                       pltpu.VMEM((_CH, 128), _F32),
                       pltpu.VMEM((_CH, 128), _F32),
                       pltpu.SemaphoreType.DMA,
                       pltpu.SemaphoreType.DMA,
                       pltpu.SemaphoreType.DMA,
                       pltpu.SemaphoreType.DMA,
                       pltpu.VMEM_SHARED((N, 128), _F32)])
    def k(ma_h, mb_h, dst_h, z_h, out_a, out_b, idx0, idx1, buf0, buf1,
          si0, sb0, si1, sb1, acc):
        cid = lax.axis_index("c")
        sid = lax.axis_index("s")
        base = sid * per_t

        @pl.when(sid == 0)
        def _():
            pltpu.sync_copy(z_h, acc)

        plsc.subcore_barrier()

        def start(i, idx, buf, si, sb):
            off = base + i * _CH
            pltpu.async_copy(dst_h.at[pl.ds(off, _CH)], idx, si)

            @pl.when(cid == 0)
            def _():
                pltpu.async_copy(ma_h.at[pl.ds(off, _CH)], buf, sb)

            @pl.when(cid == 1)
            def _():
                pltpu.async_copy(mb_h.at[pl.ds(off, _CH)], buf, sb)

        def drain_and_scatter(idx, buf, si, sb):
            pltpu.make_async_copy(dst_h.at[pl.ds(base, _CH)], idx, si).wait()
            pltpu.make_async_copy(ma_h.at[pl.ds(base, _CH)], buf, sb).wait()
            pltpu.sync_copy(buf, acc.at[idx], add=True)

        start(0, idx0, buf0, si0, sb0)

        def body(j, carry):
            i = j * 2
            start(i + 1, idx1, buf1, si1, sb1)
            drain_and_scatter(idx0, buf0, si0, sb0)

            @pl.when(i + 2 < n_chunks)
            def _():
                start(i + 2, idx0, buf0, si0, sb0)

            drain_and_scatter(idx1, buf1, si1, sb1)
            return carry

        lax.fori_loop(0, n_chunks // 2, body, 0)
        plsc.subcore_barrier()

        @pl.when((sid == 0) & (cid == 0))
        def _():
            pltpu.sync_copy(acc, out_a)

        @pl.when((sid == 0) & (cid == 1))
        def _():
            pltpu.sync_copy(acc, out_b)

    return k(msg_a, msg_b, dst, zeros)


def kernel(node_s, node_v, edge_s, edge_v, params, edge_index):
    N = node_s.shape[0]
    E = edge_s.shape[0]
    src = edge_index[0]
    dst = edge_index[1]
    nvt = node_v.swapaxes(1, 2).reshape(N, 48)
    vrow = jnp.concatenate([nvt, jnp.zeros((N, 80), _F32)], axis=1)
    sbits = lax.bitcast_convert_type(
        node_s.astype(jnp.bfloat16), jnp.uint16).astype(jnp.uint32)
    vbits = lax.bitcast_convert_type(
        vrow.astype(jnp.bfloat16), jnp.uint16).astype(jnp.uint32)
    packed = lax.bitcast_convert_type((sbits << 16) | vbits, _F32)
    evs = [edge_v[:, 0, c].reshape(1, E) for c in range(3)]

    K = 10
    Ec = E // K
    mw = _msg_weights(params)
    zeros = jnp.zeros((N, 128), _F32)
    sums_list, sumvc_list = [], []
    for k in range(K):
        lo, hi = k * Ec, (k + 1) * Ec
        gsrc, gdst = _sc_gather(packed, src[lo:hi], dst[lo:hi])
        msg_s, msg_vc = _run_msg(gsrc, gdst, edge_s[lo:hi],
                                 [e[:, lo:hi] for e in evs], mw)
        sk, vk = _sc_scatter_pair(msg_s, msg_vc, dst[lo:hi], zeros)
        sums_list.append(sk)
        sumvc_list.append(vk)
    s_out, v48 = _run_node(sums_list, sumvc_list, node_s, nvt,
                           _node_weights(params))
    v_out = v48.reshape(N, 3, 16).swapaxes(1, 2)
    return s_out, v_out
